# baseline (speedup 1.0000x reference)
import functools

import jax
import jax.numpy as jnp
from jax import lax
from jax.experimental import pallas as pl
from jax.experimental.pallas import tpu as pltpu
from jax.experimental.pallas import tpu_sc as plsc

N = 10000
E = 320000
D = 128
NCLS = 40

NC, NS = 2, 16
NW = NC * NS
EPT = E // NW
BATCH = 80
NB = EPT // BATCH
NPAD = 10240
RPT = NPAD // NS

R = 400
GRID = N // R

_mesh = plsc.VectorSubcoreMesh(core_axis_name="c", subcore_axis_name="s",
                               num_cores=NC, num_subcores=NS)


NV = NPAD // 16
NCHK = RPT // 16


def _deg_body(src_hbm, dst_hbm, deg_hbm,
              co_v, ci_v, src_v, dst_v, comb_v, res_v, co_st, ci_st):
    c = lax.axis_index("c")
    s = lax.axis_index("s")
    wid = s * NC + c
    zeros = jnp.zeros((16,), jnp.float32)
    ones = jnp.ones((16,), jnp.float32)

    def zero_body(i, carry):
        co_v[pl.ds(i * 16, 16)] = zeros
        ci_v[pl.ds(i * 16, 16)] = zeros
        return carry

    lax.fori_loop(0, NV, zero_body, 0)
    pltpu.sync_copy(src_hbm.at[wid], src_v)
    pltpu.sync_copy(dst_hbm.at[wid], dst_v)

    def count_body(j, carry):
        sv = src_v[pl.ds(j * 16, 16)]
        dv = dst_v[pl.ds(j * 16, 16)]
        plsc.addupdate_scatter(co_v, [sv], ones)
        plsc.addupdate_scatter(ci_v, [dv], ones)
        return carry

    lax.fori_loop(0, EPT // 16, count_body, 0)
    pltpu.sync_copy(co_v, co_st.at[s])
    pltpu.sync_copy(ci_v, ci_st.at[s])
    plsc.subcore_barrier()

    for a, st in ((0, co_st), (1, ci_st)):
        pltpu.sync_copy(st.at[:, pl.ds(s * RPT, RPT)], comb_v)

        def sum_body(k, carry):
            acc = comb_v[0, pl.ds(k * 16, 16)]
            for r in range(1, NS):
                acc = acc + comb_v[r, pl.ds(k * 16, 16)]
            res_v[k // 8, pl.ds((k % 8) * 16, 16)] = acc
            return carry

        lax.fori_loop(0, NCHK, sum_body, 0)
        pltpu.sync_copy(res_v, deg_hbm.at[c, a, s, pl.ds(0, RPT // 128)])


_deg_call = pl.kernel(
    _deg_body,
    out_type=jax.ShapeDtypeStruct((NC, 2, NS, 8, 128), jnp.float32),
    mesh=_mesh,
    scratch_types=[
        pltpu.VMEM((NPAD,), jnp.float32),
        pltpu.VMEM((NPAD,), jnp.float32),
        pltpu.VMEM((EPT,), jnp.int32),
        pltpu.VMEM((EPT,), jnp.int32),
        pltpu.VMEM((NS, RPT), jnp.float32),
        pltpu.VMEM((RPT // 128, 128), jnp.float32),
        pltpu.VMEM_SHARED((NS, NPAD), jnp.float32),
        pltpu.VMEM_SHARED((NS, NPAD), jnp.float32),
    ],
    compiler_params=pltpu.CompilerParams(needs_layout_passes=False),
)


IG = 5
NIG = NB // IG
NRB = 3


def _agg_body(y_hbm, src_hbm, dst_hbm, zeros_hbm, out_hbm,
              agg_sh, src_v, dst_v, rows_v, isem, gsem, ssem):
    c = lax.axis_index("c")
    s = lax.axis_index("s")
    wid = s * NC + c
    pltpu.sync_copy(zeros_hbm, agg_sh.at[pl.ds(s * RPT, RPT)])

    def fire_idx(g, q):
        pltpu.async_copy(src_hbm.at[wid, g], src_v.at[q], isem.at[q])
        pltpu.async_copy(dst_hbm.at[wid, g], dst_v.at[q], isem.at[q])

    def wait_idx(g, q):
        pltpu.make_async_copy(src_hbm.at[wid, g], src_v.at[q],
                              isem.at[q]).wait()
        pltpu.make_async_copy(dst_hbm.at[wid, g], dst_v.at[q],
                              isem.at[q]).wait()

    def fire_gather(jb, q, i):
        b = lax.rem(jb, NRB)
        pltpu.async_copy(y_hbm.at[src_v.at[q, i]], rows_v.at[b], gsem.at[b])

    def wait_gather(jb, q, i):
        b = lax.rem(jb, NRB)
        pltpu.make_async_copy(y_hbm.at[src_v.at[q, i]], rows_v.at[b],
                              gsem.at[b]).wait()

    def fire_scatter(jb, q, i):
        b = lax.rem(jb, NRB)
        pltpu.async_copy(rows_v.at[b], agg_sh.at[dst_v.at[q, i]],
                         ssem.at[b], add=True)

    def wait_scatter(jb, q, i):
        b = lax.rem(jb, NRB)
        pltpu.make_async_copy(rows_v.at[b], agg_sh.at[dst_v.at[q, i]],
                              ssem.at[b]).wait()

    fire_idx(0, 0)
    fire_idx(1, 1)
    wait_idx(0, 0)
    plsc.subcore_barrier()
    fire_gather(0, 0, 0)
    fire_gather(1, 0, 1)
    fire_gather(2, 0, 2)

    def body(g, carry):
        q = lax.rem(g, 2)
        for i in range(IG):
            j = g * IG + i
            if i == 0:
                @pl.when(g >= 1)
                def _():
                    wait_scatter(j - 1, 1 - q, IG - 1)
                    fire_gather(j + 2, q, 2)
            elif i <= 2:
                wait_scatter(j - 1, q, i - 1)
                fire_gather(j + 2, q, i + 2)
            else:
                wait_scatter(j - 1, q, i - 1)

                @pl.when(g < NIG - 1)
                def _(i=i):
                    fire_gather(j + 2, 1 - q, i - 3)

            if i == 1:
                @pl.when((g >= 1) & (g < NIG - 1))
                def _():
                    fire_idx(g + 1, 1 - q)
            if i == 2:
                @pl.when(g < NIG - 1)
                def _():
                    wait_idx(g + 1, 1 - q)

            wait_gather(j, q, i)
            fire_scatter(j, q, i)
        return carry

    lax.fori_loop(0, NIG, body, 0)
    wait_scatter(NB - 1, (NIG - 1) % 2, IG - 1)
    plsc.subcore_barrier()
    pltpu.sync_copy(agg_sh.at[pl.ds(s * RPT, RPT)],
                    out_hbm.at[c, pl.ds(s * RPT, RPT)])


_agg_call = pl.kernel(
    _agg_body,
    out_type=jax.ShapeDtypeStruct((NC, NPAD, D), jnp.float32),
    mesh=_mesh,
    scratch_types=[
        pltpu.VMEM_SHARED((NPAD, D), jnp.float32),
        pltpu.VMEM((2, IG, BATCH), jnp.int32),
        pltpu.VMEM((2, IG, BATCH), jnp.int32),
        pltpu.VMEM((NRB, BATCH, D), jnp.float32),
        pltpu.SemaphoreType.DMA((2,)),
        pltpu.SemaphoreType.DMA((NRB,)),
        pltpu.SemaphoreType.DMA((NRB,)),
    ],
)


def _prep_kernel(feats_ref, w_ref, deg_ref, y_ref, ns_ref, nd_ref):
    do = deg_ref[0, 0] + deg_ref[1, 0]
    di = deg_ref[0, 1] + deg_ref[1, 1]
    ns = lax.rsqrt(jnp.maximum(do, 1.0))
    nd = lax.rsqrt(jnp.maximum(di, 1.0))
    z = jnp.dot(feats_ref[...], w_ref[...], preferred_element_type=jnp.float32)
    y_ref[...] = z * ns
    ns_ref[...] = ns
    nd_ref[...] = nd


_prep_call = pl.pallas_call(
    _prep_kernel,
    grid=(GRID,),
    in_specs=[
        pl.BlockSpec((R, D), lambda i: (i, 0)),
        pl.BlockSpec((D, D), lambda i: (0, 0)),
        pl.BlockSpec((NC, 2, R, 1), lambda i: (0, 0, i, 0)),
    ],
    out_specs=[
        pl.BlockSpec((R, D), lambda i: (i, 0)),
        pl.BlockSpec((R, 1), lambda i: (i, 0)),
        pl.BlockSpec((R, 1), lambda i: (i, 0)),
    ],
    out_shape=[
        jax.ShapeDtypeStruct((N, D), jnp.float32),
        jax.ShapeDtypeStruct((N, 1), jnp.float32),
        jax.ShapeDtypeStruct((N, 1), jnp.float32),
    ],
)


def _layer_kernel(agg_ref, w_ref, nd_ref, ns_ref, y_ref):
    a = agg_ref[0] + agg_ref[1]
    h = jnp.maximum(a * nd_ref[...], 0.0) * ns_ref[...]
    y_ref[...] = jnp.dot(h, w_ref[...], preferred_element_type=jnp.float32)


_layer_call = pl.pallas_call(
    _layer_kernel,
    grid=(GRID,),
    in_specs=[
        pl.BlockSpec((NC, R, D), lambda i: (0, i, 0)),
        pl.BlockSpec((D, D), lambda i: (0, 0)),
        pl.BlockSpec((R, 1), lambda i: (i, 0)),
        pl.BlockSpec((R, 1), lambda i: (i, 0)),
    ],
    out_specs=pl.BlockSpec((R, D), lambda i: (i, 0)),
    out_shape=jax.ShapeDtypeStruct((N, D), jnp.float32),
)


def _final_kernel(agg_ref, nd_ref, wc_ref, out_ref, acc_ref):
    i = pl.program_id(0)
    a = agg_ref[0] + agg_ref[1]
    h = jnp.maximum(a * nd_ref[...], 0.0)
    psum = jnp.sum(h, axis=0, keepdims=True)

    @pl.when(i == 0)
    def _():
        acc_ref[...] = psum

    @pl.when(i > 0)
    def _():
        acc_ref[...] += psum

    @pl.when(i == GRID - 1)
    def _():
        pooled = acc_ref[...] * (1.0 / N)
        out_ref[...] = lax.dot_general(
            pooled, wc_ref[...], (((1,), (1,)), ((), ())),
            preferred_element_type=jnp.float32)


_final_call = pl.pallas_call(
    _final_kernel,
    grid=(GRID,),
    in_specs=[
        pl.BlockSpec((NC, R, D), lambda i: (0, i, 0)),
        pl.BlockSpec((R, 1), lambda i: (i, 0)),
        pl.BlockSpec((NCLS, D), lambda i: (0, 0)),
    ],
    out_specs=pl.BlockSpec((1, NCLS), lambda i: (0, 0)),
    out_shape=jax.ShapeDtypeStruct((1, NCLS), jnp.float32),
    scratch_shapes=[pltpu.VMEM((1, D), jnp.float32)],
)


def kernel(feats, edge_index, W1, W2, Wc):
    src = edge_index[0].reshape(NW, NIG, IG, BATCH)
    dst = edge_index[1].reshape(NW, NIG, IG, BATCH)
    z128 = jnp.zeros((RPT, D), jnp.float32)

    deg = _deg_call(edge_index[0].reshape(NW, EPT),
                    edge_index[1].reshape(NW, EPT))
    deg1 = deg[:, :, :, :RPT // 128].reshape(NC, 2, NPAD, 1)[:, :, :N]
    y1, ns, nd = _prep_call(feats, W1, deg1)
    agg1 = _agg_call(y1, src, dst, z128)
    y2 = _layer_call(agg1, W2, nd, ns)
    agg2 = _agg_call(y2, src, dst, z128)
    return _final_call(agg2, nd, Wc)


# speedup vs baseline: 1.1528x; 1.1528x over previous
import functools

import jax
import jax.numpy as jnp
from jax import lax
from jax.experimental import pallas as pl
from jax.experimental.pallas import tpu as pltpu
from jax.experimental.pallas import tpu_sc as plsc

N = 10000
E = 320000
D = 128
NCLS = 40

NC, NS = 2, 16
NW = NC * NS
EPT = E // NW
BATCH = 80
NB = EPT // BATCH
NPAD = 10240
RPT = NPAD // NS

R = 400
GRID = N // R

_mesh = plsc.VectorSubcoreMesh(core_axis_name="c", subcore_axis_name="s",
                               num_cores=NC, num_subcores=NS)


NV = NPAD // 16
NCHK = RPT // 16
EPC = E // NS
ECH = 4000


def _norm_body(idx_hbm, norm_hbm, cnt_v, idx_v, comb_v, res_v, cnt_st):
    c = lax.axis_index("c")
    s = lax.axis_index("s")
    zeros = jnp.zeros((16,), jnp.float32)
    ones = jnp.ones((16,), jnp.float32)
    iota = lax.iota(jnp.int32, 16)

    def zero_body(i, carry):
        cnt_v[pl.ds(i * 16, 16)] = zeros
        return carry

    lax.fori_loop(0, NV, zero_body, 0)
    pltpu.sync_copy(idx_hbm.at[c * NS + s], idx_v)

    def count_body(j, carry):
        iv = idx_v[pl.ds(j * 16, 16)]
        plsc.addupdate_scatter(cnt_v, [iv], ones)
        return carry

    lax.fori_loop(0, EPC // 16, count_body, 0)
    pltpu.sync_copy(cnt_v, cnt_st.at[s])
    plsc.subcore_barrier()
    pltpu.sync_copy(cnt_st.at[:, pl.ds(s * RPT, RPT)], comb_v)

    def norm_body(k, carry):
        acc = comb_v[0, pl.ds(k * 16, 16)]
        for r in range(1, NS):
            acc = acc + comb_v[r, pl.ds(k * 16, 16)]
        x = jnp.maximum(acc, 1.0)
        i = plsc.bitcast(x, jnp.int32)
        i = 0x5F3759DF - (i >> 1)
        y = plsc.bitcast(i, jnp.float32)
        for _ in range(3):
            y = y * (1.5 - 0.5 * x * y * y)
        rows = lax.rem(k, NCHK // 2) * 16 + iota
        for r in range(8):
            plsc.store_scatter(res_v, [rows, jnp.full((16,), r, jnp.int32)],
                               y)
        return carry

    for h in range(2):
        lax.fori_loop(h * (NCHK // 2), (h + 1) * (NCHK // 2), norm_body, 0)
        pltpu.sync_copy(res_v,
                        norm_hbm.at[c, pl.ds(s * RPT + h * (RPT // 2),
                                             RPT // 2)])


_norm_call = pl.kernel(
    _norm_body,
    out_type=jax.ShapeDtypeStruct((NC, NPAD, 8), jnp.float32),
    mesh=_mesh,
    scratch_types=[
        pltpu.VMEM((NPAD,), jnp.float32),
        pltpu.VMEM((EPC,), jnp.int32),
        pltpu.VMEM((NS, RPT), jnp.float32),
        pltpu.VMEM((RPT // 2, 8), jnp.float32),
        pltpu.VMEM_SHARED((NS, NPAD), jnp.float32),
    ],
    compiler_params=pltpu.CompilerParams(needs_layout_passes=False),
)


IG = 5
NIG = NB // IG
NRB = 3


def _agg_body(y_hbm, src_hbm, dst_hbm, zeros_hbm, out_hbm,
              agg_sh, src_v, dst_v, rows_v, isem, gsem, ssem):
    c = lax.axis_index("c")
    s = lax.axis_index("s")
    wid = s * NC + c
    pltpu.sync_copy(zeros_hbm, agg_sh.at[pl.ds(s * RPT, RPT)])

    def fire_idx(g, q):
        pltpu.async_copy(src_hbm.at[wid, g], src_v.at[q], isem.at[q])
        pltpu.async_copy(dst_hbm.at[wid, g], dst_v.at[q], isem.at[q])

    def wait_idx(g, q):
        pltpu.make_async_copy(src_hbm.at[wid, g], src_v.at[q],
                              isem.at[q]).wait()
        pltpu.make_async_copy(dst_hbm.at[wid, g], dst_v.at[q],
                              isem.at[q]).wait()

    def fire_gather(jb, q, i):
        b = lax.rem(jb, NRB)
        pltpu.async_copy(y_hbm.at[src_v.at[q, i]], rows_v.at[b], gsem.at[b])

    def wait_gather(jb, q, i):
        b = lax.rem(jb, NRB)
        pltpu.make_async_copy(y_hbm.at[src_v.at[q, i]], rows_v.at[b],
                              gsem.at[b]).wait()

    def fire_scatter(jb, q, i):
        b = lax.rem(jb, NRB)
        pltpu.async_copy(rows_v.at[b], agg_sh.at[dst_v.at[q, i]],
                         ssem.at[b], add=True)

    def wait_scatter(jb, q, i):
        b = lax.rem(jb, NRB)
        pltpu.make_async_copy(rows_v.at[b], agg_sh.at[dst_v.at[q, i]],
                              ssem.at[b]).wait()

    fire_idx(0, 0)
    fire_idx(1, 1)
    wait_idx(0, 0)
    plsc.subcore_barrier()
    fire_gather(0, 0, 0)
    fire_gather(1, 0, 1)
    fire_gather(2, 0, 2)

    def body(g, carry):
        q = lax.rem(g, 2)
        for i in range(IG):
            j = g * IG + i
            if i == 0:
                @pl.when(g >= 1)
                def _():
                    wait_scatter(j - 1, 1 - q, IG - 1)
                    fire_gather(j + 2, q, 2)
            elif i <= 2:
                wait_scatter(j - 1, q, i - 1)
                fire_gather(j + 2, q, i + 2)
            else:
                wait_scatter(j - 1, q, i - 1)

                @pl.when(g < NIG - 1)
                def _(i=i):
                    fire_gather(j + 2, 1 - q, i - 3)

            if i == 1:
                @pl.when((g >= 1) & (g < NIG - 1))
                def _():
                    fire_idx(g + 1, 1 - q)
            if i == 2:
                @pl.when(g < NIG - 1)
                def _():
                    wait_idx(g + 1, 1 - q)

            wait_gather(j, q, i)
            fire_scatter(j, q, i)
        return carry

    lax.fori_loop(0, NIG, body, 0)
    wait_scatter(NB - 1, (NIG - 1) % 2, IG - 1)
    plsc.subcore_barrier()
    pltpu.sync_copy(agg_sh.at[pl.ds(s * RPT, RPT)],
                    out_hbm.at[c, pl.ds(s * RPT, RPT)])


_agg_call = pl.kernel(
    _agg_body,
    out_type=jax.ShapeDtypeStruct((NC, NPAD, D), jnp.float32),
    mesh=_mesh,
    scratch_types=[
        pltpu.VMEM_SHARED((NPAD, D), jnp.float32),
        pltpu.VMEM((2, IG, BATCH), jnp.int32),
        pltpu.VMEM((2, IG, BATCH), jnp.int32),
        pltpu.VMEM((NRB, BATCH, D), jnp.float32),
        pltpu.SemaphoreType.DMA((2,)),
        pltpu.SemaphoreType.DMA((NRB,)),
        pltpu.SemaphoreType.DMA((NRB,)),
    ],
)


def _prep_kernel(feats_ref, w_ref, norm_ref, y_ref):
    ns = norm_ref[0, :, 0:1]
    z = jnp.dot(feats_ref[...], w_ref[...], preferred_element_type=jnp.float32)
    y_ref[...] = z * ns


_prep_call = pl.pallas_call(
    _prep_kernel,
    grid=(GRID,),
    in_specs=[
        pl.BlockSpec((R, D), lambda i: (i, 0)),
        pl.BlockSpec((D, D), lambda i: (0, 0)),
        pl.BlockSpec((NC, R, 8), lambda i: (0, i, 0)),
    ],
    out_specs=pl.BlockSpec((R, D), lambda i: (i, 0)),
    out_shape=jax.ShapeDtypeStruct((N, D), jnp.float32),
)


def _layer_kernel(agg_ref, w_ref, norm_ref, y_ref):
    nd = norm_ref[1, :, 0:1]
    ns = norm_ref[0, :, 0:1]
    a = agg_ref[0] + agg_ref[1]
    h = jnp.maximum(a * nd, 0.0) * ns
    y_ref[...] = jnp.dot(h, w_ref[...], preferred_element_type=jnp.float32)


_layer_call = pl.pallas_call(
    _layer_kernel,
    grid=(GRID,),
    in_specs=[
        pl.BlockSpec((NC, R, D), lambda i: (0, i, 0)),
        pl.BlockSpec((D, D), lambda i: (0, 0)),
        pl.BlockSpec((NC, R, 8), lambda i: (0, i, 0)),
    ],
    out_specs=pl.BlockSpec((R, D), lambda i: (i, 0)),
    out_shape=jax.ShapeDtypeStruct((N, D), jnp.float32),
)


def _final_kernel(agg_ref, norm_ref, wc_ref, out_ref, acc_ref):
    i = pl.program_id(0)
    nd = norm_ref[1, :, 0:1]
    a = agg_ref[0] + agg_ref[1]
    h = jnp.maximum(a * nd, 0.0)
    psum = jnp.sum(h, axis=0, keepdims=True)

    @pl.when(i == 0)
    def _():
        acc_ref[...] = psum

    @pl.when(i > 0)
    def _():
        acc_ref[...] += psum

    @pl.when(i == GRID - 1)
    def _():
        pooled = acc_ref[...] * (1.0 / N)
        out_ref[...] = lax.dot_general(
            pooled, wc_ref[...], (((1,), (1,)), ((), ())),
            preferred_element_type=jnp.float32)


_final_call = pl.pallas_call(
    _final_kernel,
    grid=(GRID,),
    in_specs=[
        pl.BlockSpec((NC, R, D), lambda i: (0, i, 0)),
        pl.BlockSpec((NC, R, 8), lambda i: (0, i, 0)),
        pl.BlockSpec((NCLS, D), lambda i: (0, 0)),
    ],
    out_specs=pl.BlockSpec((1, NCLS), lambda i: (0, 0)),
    out_shape=jax.ShapeDtypeStruct((1, NCLS), jnp.float32),
    scratch_shapes=[pltpu.VMEM((1, D), jnp.float32)],
)


def kernel(feats, edge_index, W1, W2, Wc):
    src = edge_index[0].reshape(NW, NIG, IG, BATCH)
    dst = edge_index[1].reshape(NW, NIG, IG, BATCH)
    z128 = jnp.zeros((RPT, D), jnp.float32)

    norm = _norm_call(edge_index.reshape(NC * NS, EPC))
    y1 = _prep_call(feats, W1, norm)
    agg1 = _agg_call(y1, src, dst, z128)
    y2 = _layer_call(agg1, W2, norm)
    agg2 = _agg_call(y2, src, dst, z128)
    return _final_call(agg2, norm, Wc)


# speedup vs baseline: 1.2586x; 1.0918x over previous
import functools

import jax
import jax.numpy as jnp
from jax import lax
from jax.experimental import pallas as pl
from jax.experimental.pallas import tpu as pltpu
from jax.experimental.pallas import tpu_sc as plsc

N = 10000
E = 320000
D = 128
NCLS = 40

NC, NS = 2, 16
NW = NC * NS
EPT = E // NW
BATCH = 80
NB = EPT // BATCH
NPAD = 10240
RPT = NPAD // NS

R = 1000
GRID = N // R

_mesh = plsc.VectorSubcoreMesh(core_axis_name="c", subcore_axis_name="s",
                               num_cores=NC, num_subcores=NS)


NV = NPAD // 16
NCHK = RPT // 16
EPC = E // NS
ECH = 4000


def _norm_body(idx_hbm, norm_hbm, cnt_v, idx_v, comb_v, res_v, cnt_st):
    c = lax.axis_index("c")
    s = lax.axis_index("s")
    zeros = jnp.zeros((16,), jnp.float32)
    ones = jnp.ones((16,), jnp.float32)
    iota = lax.iota(jnp.int32, 16)

    def zero_body(i, carry):
        cnt_v[pl.ds(i * 16, 16)] = zeros
        return carry

    lax.fori_loop(0, NV, zero_body, 0)
    pltpu.sync_copy(idx_hbm.at[c * NS + s], idx_v)

    def count_body(j, carry):
        iv = idx_v[pl.ds(j * 16, 16)]
        plsc.addupdate_scatter(cnt_v, [iv], ones)
        return carry

    lax.fori_loop(0, EPC // 16, count_body, 0)
    pltpu.sync_copy(cnt_v, cnt_st.at[s])
    plsc.subcore_barrier()
    pltpu.sync_copy(cnt_st.at[:, pl.ds(s * RPT, RPT)], comb_v)

    def norm_body(k, carry):
        acc = comb_v[0, pl.ds(k * 16, 16)]
        for r in range(1, NS):
            acc = acc + comb_v[r, pl.ds(k * 16, 16)]
        x = jnp.maximum(acc, 1.0)
        i = plsc.bitcast(x, jnp.int32)
        i = 0x5F3759DF - (i >> 1)
        y = plsc.bitcast(i, jnp.float32)
        for _ in range(3):
            y = y * (1.5 - 0.5 * x * y * y)
        rows = lax.rem(k, NCHK // 2) * 16 + iota
        for r in range(8):
            plsc.store_scatter(res_v, [rows, jnp.full((16,), r, jnp.int32)],
                               y)
        return carry

    for h in range(2):
        lax.fori_loop(h * (NCHK // 2), (h + 1) * (NCHK // 2), norm_body, 0)
        pltpu.sync_copy(res_v,
                        norm_hbm.at[c, pl.ds(s * RPT + h * (RPT // 2),
                                             RPT // 2)])


_norm_call = pl.kernel(
    _norm_body,
    out_type=jax.ShapeDtypeStruct((NC, NPAD, 8), jnp.float32),
    mesh=_mesh,
    scratch_types=[
        pltpu.VMEM((NPAD,), jnp.float32),
        pltpu.VMEM((EPC,), jnp.int32),
        pltpu.VMEM((NS, RPT), jnp.float32),
        pltpu.VMEM((RPT // 2, 8), jnp.float32),
        pltpu.VMEM_SHARED((NS, NPAD), jnp.float32),
    ],
    compiler_params=pltpu.CompilerParams(needs_layout_passes=False),
)


IG = 5
NIG = NB // IG
NRB = 3


def _agg_body(y_hbm, ei_hbm, zeros_hbm, out_hbm,
              agg_sh, src_v, dst_v, rows_v, isem, gsem, ssem):
    c = lax.axis_index("c")
    s = lax.axis_index("s")
    wid = s * NC + c
    pltpu.sync_copy(zeros_hbm, agg_sh.at[pl.ds(s * RPT, RPT)])

    def fire_idx(g, q):
        pltpu.async_copy(ei_hbm.at[0, wid, g], src_v.at[q], isem.at[q])
        pltpu.async_copy(ei_hbm.at[1, wid, g], dst_v.at[q], isem.at[q])

    def wait_idx(g, q):
        pltpu.make_async_copy(ei_hbm.at[0, wid, g], src_v.at[q],
                              isem.at[q]).wait()
        pltpu.make_async_copy(ei_hbm.at[1, wid, g], dst_v.at[q],
                              isem.at[q]).wait()

    def fire_gather(jb, q, i):
        b = lax.rem(jb, NRB)
        pltpu.async_copy(y_hbm.at[src_v.at[q, i]], rows_v.at[b], gsem.at[b])

    def wait_gather(jb, q, i):
        b = lax.rem(jb, NRB)
        pltpu.make_async_copy(y_hbm.at[src_v.at[q, i]], rows_v.at[b],
                              gsem.at[b]).wait()

    def fire_scatter(jb, q, i):
        b = lax.rem(jb, NRB)
        pltpu.async_copy(rows_v.at[b], agg_sh.at[dst_v.at[q, i]],
                         ssem.at[b], add=True)

    def wait_scatter(jb, q, i):
        b = lax.rem(jb, NRB)
        pltpu.make_async_copy(rows_v.at[b], agg_sh.at[dst_v.at[q, i]],
                              ssem.at[b]).wait()

    fire_idx(0, 0)
    fire_idx(1, 1)
    wait_idx(0, 0)
    plsc.subcore_barrier()
    fire_gather(0, 0, 0)
    fire_gather(1, 0, 1)
    fire_gather(2, 0, 2)

    def body(g, carry):
        q = lax.rem(g, 2)
        for i in range(IG):
            j = g * IG + i
            if i == 0:
                @pl.when(g >= 1)
                def _():
                    wait_scatter(j - 1, 1 - q, IG - 1)
                    fire_gather(j + 2, q, 2)
            elif i <= 2:
                wait_scatter(j - 1, q, i - 1)
                fire_gather(j + 2, q, i + 2)
            else:
                wait_scatter(j - 1, q, i - 1)

                @pl.when(g < NIG - 1)
                def _(i=i):
                    fire_gather(j + 2, 1 - q, i - 3)

            if i == 1:
                @pl.when((g >= 1) & (g < NIG - 1))
                def _():
                    fire_idx(g + 1, 1 - q)
            if i == 2:
                @pl.when(g < NIG - 1)
                def _():
                    wait_idx(g + 1, 1 - q)

            wait_gather(j, q, i)
            fire_scatter(j, q, i)
        return carry

    lax.fori_loop(0, NIG, body, 0)
    wait_scatter(NB - 1, (NIG - 1) % 2, IG - 1)
    plsc.subcore_barrier()
    pltpu.sync_copy(agg_sh.at[pl.ds(s * RPT, RPT)],
                    out_hbm.at[c, pl.ds(s * RPT, RPT)])


_agg_call = pl.kernel(
    _agg_body,
    out_type=jax.ShapeDtypeStruct((NC, NPAD, D), jnp.float32),
    mesh=_mesh,
    scratch_types=[
        pltpu.VMEM_SHARED((NPAD, D), jnp.float32),
        pltpu.VMEM((2, IG, BATCH), jnp.int32),
        pltpu.VMEM((2, IG, BATCH), jnp.int32),
        pltpu.VMEM((NRB, BATCH, D), jnp.float32),
        pltpu.SemaphoreType.DMA((2,)),
        pltpu.SemaphoreType.DMA((NRB,)),
        pltpu.SemaphoreType.DMA((NRB,)),
    ],
)


def _prep_kernel(feats_ref, w_ref, norm_ref, y_ref):
    ns = norm_ref[0, :, 0:1]
    z = jnp.dot(feats_ref[...], w_ref[...], preferred_element_type=jnp.float32)
    y_ref[...] = z * ns


_prep_call = pl.pallas_call(
    _prep_kernel,
    grid=(GRID,),
    in_specs=[
        pl.BlockSpec((R, D), lambda i: (i, 0)),
        pl.BlockSpec((D, D), lambda i: (0, 0)),
        pl.BlockSpec((NC, R, 8), lambda i: (0, i, 0)),
    ],
    out_specs=pl.BlockSpec((R, D), lambda i: (i, 0)),
    out_shape=jax.ShapeDtypeStruct((N, D), jnp.float32),
)


def _layer_kernel(agg_ref, w_ref, norm_ref, y_ref):
    nd = norm_ref[1, :, 0:1]
    ns = norm_ref[0, :, 0:1]
    a = agg_ref[0] + agg_ref[1]
    h = jnp.maximum(a * nd, 0.0) * ns
    y_ref[...] = jnp.dot(h, w_ref[...], preferred_element_type=jnp.float32)


_layer_call = pl.pallas_call(
    _layer_kernel,
    grid=(GRID,),
    in_specs=[
        pl.BlockSpec((NC, R, D), lambda i: (0, i, 0)),
        pl.BlockSpec((D, D), lambda i: (0, 0)),
        pl.BlockSpec((NC, R, 8), lambda i: (0, i, 0)),
    ],
    out_specs=pl.BlockSpec((R, D), lambda i: (i, 0)),
    out_shape=jax.ShapeDtypeStruct((N, D), jnp.float32),
)


def _final_kernel(agg_ref, norm_ref, wc_ref, out_ref, acc_ref):
    i = pl.program_id(0)
    nd = norm_ref[1, :, 0:1]
    a = agg_ref[0] + agg_ref[1]
    h = jnp.maximum(a * nd, 0.0)
    psum = jnp.sum(h, axis=0, keepdims=True)

    @pl.when(i == 0)
    def _():
        acc_ref[...] = psum

    @pl.when(i > 0)
    def _():
        acc_ref[...] += psum

    @pl.when(i == GRID - 1)
    def _():
        pooled = acc_ref[...] * (1.0 / N)
        out_ref[...] = lax.dot_general(
            pooled, wc_ref[...], (((1,), (1,)), ((), ())),
            preferred_element_type=jnp.float32)


_final_call = pl.pallas_call(
    _final_kernel,
    grid=(GRID,),
    in_specs=[
        pl.BlockSpec((NC, R, D), lambda i: (0, i, 0)),
        pl.BlockSpec((NC, R, 8), lambda i: (0, i, 0)),
        pl.BlockSpec((NCLS, D), lambda i: (0, 0)),
    ],
    out_specs=pl.BlockSpec((1, NCLS), lambda i: (0, 0)),
    out_shape=jax.ShapeDtypeStruct((1, NCLS), jnp.float32),
    scratch_shapes=[pltpu.VMEM((1, D), jnp.float32)],
)


def kernel(feats, edge_index, W1, W2, Wc):
    ei5 = edge_index.reshape(2, NW, NIG, IG, BATCH)
    z128 = jnp.zeros((RPT, D), jnp.float32)

    norm = _norm_call(edge_index.reshape(NC * NS, EPC))
    y1 = _prep_call(feats, W1, norm)
    agg1 = _agg_call(y1, ei5, z128)
    y2 = _layer_call(agg1, W2, norm)
    agg2 = _agg_call(y2, ei5, z128)
    return _final_call(agg2, norm, Wc)
